# Initial kernel scaffold; baseline (speedup 1.0000x reference)
import jax
import jax.numpy as jnp
from jax.experimental import pallas as pl


def kernel(x_user, x_item, edge_index_u2i, edge_index_i2u, target_edge_index, W1l_u2i, b1l_u2i, W1r_u2i, W2l_u2i, b2l_u2i, W2r_u2i, W1l_i2u, b1l_i2u, W1r_i2u, W2l_i2u, b2l_i2u, W2r_i2u, Wc1, bc1, Wc2, bc2):
    raise NotImplementedError("write your pallas kernel here")


# speedup vs baseline: 2.6053x; 2.6053x over previous
import functools

import jax
import jax.numpy as jnp
from jax import lax
from jax.experimental import pallas as pl
from jax.experimental.pallas import tpu as pltpu
from jax.experimental.pallas import tpu_sc as plsc

N = 10000
NPAD = 10240
H = 128
E = 320000
ET = 100000
CW = 128
EBLK = 320
EPAD = EBLK * 8 * CW
TBLK = 100
TPAD = TBLK * 8 * CW
CNTW = 16
NSUB = 16
NROWT = NPAD // NSUB
GG = 2
GGT = 4

_f32 = jnp.float32
_i32 = jnp.int32


def _segsum_body(xa, ra, ca, xb, rb, cb, znh, sums_a, sums_b,
                 acc, idxs, idxd, rows, sem):
    c = lax.axis_index("c")
    t = lax.axis_index("s")
    r0 = t * NROWT

    pltpu.sync_copy(znh.at[pl.ds(r0, NROWT)], acc.at[pl.ds(r0, NROWT)])
    plsc.subcore_barrier()

    blocks_per_tile = EBLK // NSUB

    def _process(x_hbm, row3, col3):
        base = t * blocks_per_tile

        def _chunk(g, carry):
            b0 = base + g
            pltpu.sync_copy(row3.at[b0], idxs)
            pltpu.sync_copy(col3.at[b0], idxd)
            for hh in range(8 // GG):
                descs = [
                    pltpu.async_copy(x_hbm.at[idxs.at[hh * GG + j]],
                                     rows.at[j], sem)
                    for j in range(GG)
                ]
                for d in descs:
                    d.wait()
                for j in range(GG):
                    jj = hh * GG + j
                    pltpu.sync_copy(rows.at[j], acc.at[idxd.at[jj]], add=True)
            return carry

        lax.fori_loop(0, blocks_per_tile, _chunk, 0)

    @pl.when(c == 0)
    def _():
        _process(xa, ra, ca)

    @pl.when(c == 1)
    def _():
        _process(xb, rb, cb)

    plsc.subcore_barrier()

    @pl.when(c == 0)
    def _():
        pltpu.sync_copy(acc.at[pl.ds(r0, NROWT)], sums_a.at[pl.ds(r0, NROWT)])

    @pl.when(c == 1)
    def _():
        pltpu.sync_copy(acc.at[pl.ds(r0, NROWT)], sums_b.at[pl.ds(r0, NROWT)])


@functools.lru_cache(maxsize=None)
def _make_segsum():
    mesh = plsc.VectorSubcoreMesh(core_axis_name="c", subcore_axis_name="s",
                                  num_cores=2, num_subcores=NSUB)
    return pl.kernel(
        _segsum_body,
        out_type=[jax.ShapeDtypeStruct((NPAD, H), _f32)] * 2,
        mesh=mesh,
        scratch_types=[pltpu.VMEM_SHARED((NPAD, H), _f32),
                       pltpu.VMEM((8, CW), _i32), pltpu.VMEM((8, CW), _i32),
                       pltpu.VMEM((GG, CW, H), _f32),
                       pltpu.SemaphoreType.DMA],
        name="segsum",
    )


def _segsum(*args):
    return _make_segsum()(*args)


def _degree_body(ca3, cb3, cnt_a, cnt_b, cnth, hist, idxd, iota, sem):
    c = lax.axis_index("c")
    t = lax.axis_index("s")

    def _zero(i, carry):
        r = i // 8
        k = i % 8
        hist[r, pl.ds(k * 16, 16)] = jnp.zeros((16,), _f32)
        return carry
    lax.fori_loop(0, CW * 8, _zero, 0)
    for k in range(8):
        iota[0, pl.ds(k * 16, 16)] = lax.iota(_i32, 16) + (k * 16)
    pltpu.sync_copy(hist.at[pl.ds(0, 8)], cnth.at[pl.ds(t * 8, 8)])
    plsc.subcore_barrier()

    blocks_per_tile = EBLK // NSUB

    def _process(col3):
        base = t * blocks_per_tile

        def _chunk(g, carry):
            b0 = base + g
            pltpu.sync_copy(col3.at[b0], idxd)
            for j in range(8):
                for k in range(8):
                    v = idxd[j, pl.ds(k * 16, 16)]
                    plsc.addupdate_scatter(
                        hist, [v >> 7, v & 127], jnp.full((16,), 1.0, _f32))
            return carry

        lax.fori_loop(0, blocks_per_tile, _chunk, 0)

    @pl.when(c == 0)
    def _():
        _process(ca3)

    @pl.when(c == 1)
    def _():
        _process(cb3)

    plsc.subcore_barrier()
    pltpu.sync_copy(hist, cnth.at[iota.at[0]], add=True)
    plsc.subcore_barrier()

    @pl.when(c == 0)
    def _():
        pltpu.sync_copy(cnth.at[pl.ds(t * 8, 8)], cnt_a.at[pl.ds(t * 8, 8)])

    @pl.when(c == 1)
    def _():
        pltpu.sync_copy(cnth.at[pl.ds(t * 8, 8)], cnt_b.at[pl.ds(t * 8, 8)])


@functools.lru_cache(maxsize=None)
def _make_degree():
    mesh = plsc.VectorSubcoreMesh(core_axis_name="c", subcore_axis_name="s",
                                  num_cores=2, num_subcores=NSUB)
    return pl.kernel(
        _degree_body,
        out_type=[jax.ShapeDtypeStruct((CW, CW), _f32)] * 2,
        mesh=mesh,
        scratch_types=[pltpu.VMEM_SHARED((CW, CW), _f32),
                       pltpu.VMEM((CW, CW), _f32),
                       pltpu.VMEM((8, CW), _i32),
                       pltpu.VMEM((1, CW), _i32),
                       pltpu.SemaphoreType.DMA],
        compiler_params=pltpu.CompilerParams(needs_layout_passes=False),
        name="degree",
    )


def _degree(*args):
    return _make_degree()(*args)


def _gather2_body(za, ia, zb, ib, ga, gb, idx, rows, sem):
    c = lax.axis_index("c")
    t = lax.axis_index("s")
    nblocks = 6 + (t < 4).astype(_i32)
    base = t * 6 + jnp.minimum(t, 4)

    def _process(z_hbm, i3, out4):
        def _chunk(g, carry):
            b0 = base + g
            pltpu.sync_copy(i3.at[b0], idx)
            for hh in range(8 // GGT):
                descs = [
                    pltpu.async_copy(z_hbm.at[idx.at[hh * GGT + j]],
                                     rows.at[j], sem)
                    for j in range(GGT)
                ]
                for d in descs:
                    d.wait()
                pltpu.sync_copy(rows, out4.at[b0, pl.ds(hh * GGT, GGT)])
            return carry

        lax.fori_loop(0, nblocks, _chunk, 0)

    @pl.when(c == 0)
    def _():
        _process(za, ia, ga)

    @pl.when(c == 1)
    def _():
        _process(zb, ib, gb)


@functools.lru_cache(maxsize=None)
def _make_gather2():
    return pl.kernel(
        _gather2_body,
        out_type=[jax.ShapeDtypeStruct((TBLK, 8, CW, H), _f32)] * 2,
        mesh=plsc.VectorSubcoreMesh(core_axis_name="c", subcore_axis_name="s",
                                    num_cores=2, num_subcores=NSUB),
        scratch_types=[pltpu.VMEM((8, CW), _i32),
                       pltpu.VMEM((GGT, CW, H), _f32),
                       pltpu.SemaphoreType.DMA],
        name="gather2",
    )


def _gather2(*args):
    return _make_gather2()(*args)


def _mlin_kernel(s_ref, c_ref, xd_ref, wl_ref, bl_ref, wr_ref, o_ref, *, relu):
    cntv = jnp.maximum(c_ref[:, 0:1], 1.0)
    agg = s_ref[...] / cntv
    h = jnp.dot(agg, wl_ref[...], preferred_element_type=_f32)
    h = h + jnp.dot(xd_ref[...], wr_ref[...], preferred_element_type=_f32)
    h = h + bl_ref[...]
    o_ref[...] = jnp.maximum(h, 0.0) if relu else h


def _mean_linear(sums, cnts, x_dst, wl, bl, wr, relu):
    B = 1280
    grid = (NPAD // B,)
    return pl.pallas_call(
        functools.partial(_mlin_kernel, relu=relu),
        grid=grid,
        in_specs=[
            pl.BlockSpec((B, H), lambda i: (i, 0)),
            pl.BlockSpec((B, 1), lambda i: (i, 0)),
            pl.BlockSpec((B, H), lambda i: (i, 0)),
            pl.BlockSpec((H, H), lambda i: (0, 0)),
            pl.BlockSpec((1, H), lambda i: (0, 0)),
            pl.BlockSpec((H, H), lambda i: (0, 0)),
        ],
        out_specs=pl.BlockSpec((B, H), lambda i: (i, 0)),
        out_shape=jax.ShapeDtypeStruct((NPAD, H), _f32),
    )(sums, cnts, x_dst, wl, bl.reshape(1, H), wr)


def _cls_kernel(gu_ref, gi_ref, w1u_ref, w1i_ref, b1_ref, w2_ref, b2_ref, o_ref):
    h = jnp.dot(gu_ref[...], w1u_ref[...], preferred_element_type=_f32)
    h = h + jnp.dot(gi_ref[...], w1i_ref[...], preferred_element_type=_f32)
    h = jnp.maximum(h + b1_ref[...], 0.0)
    o_ref[...] = jnp.dot(h, w2_ref[...], preferred_element_type=_f32) + b2_ref[...]


def _classifier(gu, gi, wc1, bc1, wc2, bc2):
    B = 2048
    grid = (TPAD // B,)
    return pl.pallas_call(
        _cls_kernel,
        grid=grid,
        in_specs=[
            pl.BlockSpec((B, H), lambda i: (i, 0)),
            pl.BlockSpec((B, H), lambda i: (i, 0)),
            pl.BlockSpec((H, H), lambda i: (0, 0)),
            pl.BlockSpec((H, H), lambda i: (0, 0)),
            pl.BlockSpec((1, H), lambda i: (0, 0)),
            pl.BlockSpec((H, 1), lambda i: (0, 0)),
            pl.BlockSpec((1, 1), lambda i: (0, 0)),
        ],
        out_specs=pl.BlockSpec((B, 1), lambda i: (i, 0)),
        out_shape=jax.ShapeDtypeStruct((TPAD, 1), _f32),
    )(gu, gi, wc1[:H], wc1[H:], bc1.reshape(1, H), wc2, bc2.reshape(1, 1))


def _pad_idx(a, total, fill):
    blk = total // (8 * CW)
    return jnp.concatenate(
        [a, jnp.full((total - a.shape[0],), fill, _i32)]).reshape(blk, 8, CW)


def _pad_rows(x):
    return jnp.concatenate([x, jnp.zeros((NPAD - N, H), _f32)])


def kernel(x_user, x_item, edge_index_u2i, edge_index_i2u, target_edge_index,
           W1l_u2i, b1l_u2i, W1r_u2i, W2l_u2i, b2l_u2i, W2r_u2i,
           W1l_i2u, b1l_i2u, W1r_i2u, W2l_i2u, b2l_i2u, W2r_i2u,
           Wc1, bc1, Wc2, bc2):
    ra = _pad_idx(edge_index_i2u[0], EPAD, 0)
    ca = _pad_idx(edge_index_i2u[1], EPAD, N)
    rb = _pad_idx(edge_index_u2i[0], EPAD, 0)
    cb = _pad_idx(edge_index_u2i[1], EPAD, N)
    znh = jnp.zeros((NPAD, H), _f32)
    xu = _pad_rows(x_user)
    xi = _pad_rows(x_item)

    cu2, ci2 = _degree(ca, cb)
    cnt_u = cu2.reshape(-1)[:NPAD].reshape(NPAD, 1)
    cnt_i = ci2.reshape(-1)[:NPAD].reshape(NPAD, 1)
    sums_u, sums_i = _segsum(xi, ra, ca, xu, rb, cb, znh)
    h_u = _mean_linear(sums_u, cnt_u, xu, W1l_i2u, b1l_i2u, W1r_i2u, True)
    h_i = _mean_linear(sums_i, cnt_i, xi, W1l_u2i, b1l_u2i, W1r_u2i, True)

    q_u, q_i = _segsum(h_i, ra, ca, h_u, rb, cb, znh)
    z_u = _mean_linear(q_u, cnt_u, h_u, W2l_i2u, b2l_i2u, W2r_i2u, False)
    z_i = _mean_linear(q_i, cnt_i, h_i, W2l_u2i, b2l_u2i, W2r_u2i, False)

    rt = _pad_idx(target_edge_index[0], TPAD, 0)
    ct = _pad_idx(target_edge_index[1], TPAD, 0)
    gu4, gi4 = _gather2(z_u, rt, z_i, ct)
    out = _classifier(gu4.reshape(TPAD, H), gi4.reshape(TPAD, H),
                      Wc1, bc1, Wc2, bc2)
    return out.reshape(-1)[:ET]
